# Initial kernel scaffold; baseline (speedup 1.0000x reference)
import jax
import jax.numpy as jnp
from jax.experimental import pallas as pl


def kernel(x, edge_index, edge_attr, batch, params):
    raise NotImplementedError("write your pallas kernel here")


# speedup vs baseline: 6.1991x; 6.1991x over previous
import jax
import jax.numpy as jnp
from jax import lax
from jax.experimental import pallas as pl
from jax.experimental.pallas import tpu as pltpu
from jax.experimental.pallas import tpu_sc as plsc

N = 10000
E = 160000
NACC = 10240
EPS = 1e-7
NC, NS = 2, 16
RZ = NACC // NS


def _sc_mesh():
    return plsc.VectorSubcoreMesh(
        core_axis_name="c", subcore_axis_name="s", num_cores=NC, num_subcores=NS
    )


BE_A = 80
EPT_A = E // NS
NBLK_A = EPT_A // BE_A


def _sc_conv_a_body(xs, es, srcv, dstv, beta, out,
                    acc, srcb, dstb, xb, eb, ob, zb, bb, sem0, sem1):
    c = lax.axis_index("c")
    s = lax.axis_index("s")
    zeros16 = jnp.zeros((16,), jnp.float32)

    def zrow(r, _):
        for k in range(8):
            zb[r, pl.ds(16 * k, 16)] = zeros16
        return 0
    lax.fori_loop(0, 128, zrow, 0)

    def zcp(b, _):
        pltpu.sync_copy(zb, acc.at[pl.ds(s * RZ + b * 128, 128)])
        return 0
    lax.fori_loop(0, RZ // 128, zcp, 0)
    pltpu.sync_copy(beta, bb)
    plsc.subcore_barrier()

    def blk(bi, _):
        off = s * EPT_A + bi * BE_A
        pltpu.sync_copy(srcv.at[pl.ds(off, BE_A)], srcb)
        cpx = pltpu.async_copy(xs.at[srcb], xb, sem0)
        cpe = pltpu.async_copy(es.at[pl.ds(off, BE_A)], eb, sem1)
        pltpu.sync_copy(dstv.at[pl.ds(off, BE_A)], dstb)
        cpx.wait()
        cpe.wait()
        bv = bb[...]

        def row(j, _):
            for k in range(4):
                col = 16 * k
                xk = xb[j, pl.ds(c * 64 + col, 16)]
                ek = eb[j, pl.ds(c * 64 + col, 16)]
                m = jnp.maximum(xk + ek, 0.0) + EPS
                w = jnp.exp(bv * m)
                ob[j, pl.ds(col, 16)] = w * m
                ob[j, pl.ds(64 + col, 16)] = w
            return 0
        lax.fori_loop(0, BE_A, row, 0)
        pltpu.sync_copy(ob, acc.at[dstb], add=True)
        return 0
    lax.fori_loop(0, NBLK_A, blk, 0)
    plsc.subcore_barrier()
    pltpu.sync_copy(acc.at[pl.ds(s * RZ, RZ)],
                    out.at[pl.ds(c * NACC + s * RZ, RZ)])


def _make_sc_conv_a():
    return pl.kernel(
        _sc_conv_a_body,
        out_type=jax.ShapeDtypeStruct((NC * NACC, 128), jnp.float32),
        mesh=_sc_mesh(),
        scratch_types=[
            pltpu.VMEM_SHARED((NACC, 128), jnp.float32),
            pltpu.VMEM((BE_A,), jnp.int32),
            pltpu.VMEM((BE_A,), jnp.int32),
            pltpu.VMEM((BE_A, 128), jnp.float32),
            pltpu.VMEM((BE_A, 128), jnp.float32),
            pltpu.VMEM((BE_A, 128), jnp.float32),
            pltpu.VMEM((128, 128), jnp.float32),
            pltpu.VMEM((16,), jnp.float32),
            pltpu.SemaphoreType.DMA,
            pltpu.SemaphoreType.DMA,
        ],
    )


BE_B = 40
E2 = E // NC
EPT_B = E2 // NS
NBLK_B = EPT_B // BE_B


def _sc_conv_b_body(xs, es, srcv, dstv, beta, out,
                    acc, srcb, dstb, xb, eb, ob, zb, bb, sem0, sem1):
    c = lax.axis_index("c")
    s = lax.axis_index("s")
    zeros16 = jnp.zeros((16,), jnp.float32)

    def zrow(r, _):
        for k in range(8):
            zb[r, pl.ds(16 * k, 16)] = zeros16
        return 0
    lax.fori_loop(0, 128, zrow, 0)

    def zcp(b, _):
        pltpu.sync_copy(zb, acc.at[pl.ds(s * RZ + b * 128, 128)])
        return 0
    lax.fori_loop(0, RZ // 128, zcp, 0)
    pltpu.sync_copy(beta, bb)
    plsc.subcore_barrier()

    def blk(bi, _):
        off = c * E2 + s * EPT_B + bi * BE_B
        pltpu.sync_copy(srcv.at[pl.ds(off, BE_B)], srcb)
        cpx = pltpu.async_copy(xs.at[srcb], xb, sem0)
        cpe = pltpu.async_copy(es.at[pl.ds(off, BE_B)], eb, sem1)
        pltpu.sync_copy(dstv.at[pl.ds(off, BE_B)], dstb)
        cpx.wait()
        cpe.wait()
        bv = bb[...]

        def row(j, _):
            f = xb[j, pl.ds(64, 16)]
            for k in range(4):
                col = 16 * k
                xk = xb[j, pl.ds(col, 16)]
                ek = eb[j, pl.ds(col, 16)]
                m = jnp.maximum(xk + ek, 0.0) + EPS
                wf = jnp.exp(bv * m) * f
                ob[j, pl.ds(col, 16)] = wf * m
                ob[j, pl.ds(64 + col, 16)] = wf
            return 0
        lax.fori_loop(0, BE_B, row, 0)
        pltpu.sync_copy(ob, acc.at[dstb], add=True)
        return 0
    lax.fori_loop(0, NBLK_B, blk, 0)
    plsc.subcore_barrier()
    pltpu.sync_copy(acc.at[pl.ds(s * RZ, RZ)],
                    out.at[pl.ds(c * NACC + s * RZ, RZ)])


def _make_sc_conv_b():
    return pl.kernel(
        _sc_conv_b_body,
        out_type=jax.ShapeDtypeStruct((NC * NACC, 128), jnp.float32),
        mesh=_sc_mesh(),
        scratch_types=[
            pltpu.VMEM_SHARED((NACC, 128), jnp.float32),
            pltpu.VMEM((BE_B,), jnp.int32),
            pltpu.VMEM((BE_B,), jnp.int32),
            pltpu.VMEM((BE_B, 128), jnp.float32),
            pltpu.VMEM((BE_B, 128), jnp.float32),
            pltpu.VMEM((BE_B, 128), jnp.float32),
            pltpu.VMEM((128, 128), jnp.float32),
            pltpu.VMEM((16,), jnp.float32),
            pltpu.SemaphoreType.DMA,
            pltpu.SemaphoreType.DMA,
        ],
    )


_sc_conv_a = _make_sc_conv_a()
_sc_conv_b = _make_sc_conv_b()


def _edge_feats(ea, w):
    R, din16 = ea.shape
    BR = 2000

    def body(ea_ref, w_ref, o_ref):
        o_ref[...] = jnp.dot(ea_ref[...], w_ref[...],
                             preferred_element_type=jnp.float32)

    return pl.pallas_call(
        body,
        grid=(R // BR,),
        in_specs=[
            pl.BlockSpec((BR, din16), lambda j: (j, 0)),
            pl.BlockSpec((din16, 128), lambda j: (0, 0)),
        ],
        out_specs=pl.BlockSpec((BR, 128), lambda j: (j, 0)),
        out_shape=jax.ShapeDtypeStruct((R, 128), jnp.float32),
    )(ea, w)


def _mlp_score(plane, accs, w1, b1, w2, b2, wn, alv, din, dh):
    BN = 400
    G = N // BN

    def body(x_ref, a0, a1, w1r, b1r, w2r, b2r, wnr, ar, h_ref, sm_ref):
        if din == 128:
            x = x_ref[...]
            ws = jnp.concatenate((a0[0][:, :64], a1[0][:, :64]), axis=1)
            ss = jnp.concatenate((a0[0][:, 64:], a1[0][:, 64:]), axis=1)
        else:
            x = x_ref[...][:, :64]
            ws = a0[0][:, :64] + a1[0][:, :64]
            ss = a0[0][:, 64:] + a1[0][:, 64:]
        h = x + ws / (ss + 1e-16)
        h1 = jnp.maximum(
            jnp.dot(h, w1r[...], preferred_element_type=jnp.float32) + b1r[...],
            0.0)
        h2 = jnp.dot(h1, w2r[...], preferred_element_type=jnp.float32) + b2r[...]
        h_ref[...] = h2
        sc = jnp.tanh(jnp.dot(h2, wnr[...], preferred_element_type=jnp.float32))
        sm_ref[...] = jnp.where(ar[...] > 0.0, sc, -2.0)

    return pl.pallas_call(
        body,
        grid=(G,),
        in_specs=[
            pl.BlockSpec((BN, 128), lambda j: (j, 0)),
            pl.BlockSpec((1, BN, 128), lambda j: (0, j, 0)),
            pl.BlockSpec((1, BN, 128), lambda j: (1, j, 0)),
            pl.BlockSpec((din, dh), lambda j: (0, 0)),
            pl.BlockSpec((1, dh), lambda j: (0, 0)),
            pl.BlockSpec((dh, 128), lambda j: (0, 0)),
            pl.BlockSpec((1, 128), lambda j: (0, 0)),
            pl.BlockSpec((128, 1), lambda j: (0, 0)),
            pl.BlockSpec((BN, 1), lambda j: (j, 0)),
        ],
        out_specs=[
            pl.BlockSpec((BN, 128), lambda j: (j, 0)),
            pl.BlockSpec((BN, 1), lambda j: (j, 0)),
        ],
        out_shape=[
            jax.ShapeDtypeStruct((N, 128), jnp.float32),
            jax.ShapeDtypeStruct((N, 1), jnp.float32),
        ],
    )(plane, accs, accs, w1, b1, w2, b2, wn, alv)


def _pool_mask(sm80, k):
    def body(sm_ref, keep_ref, scl_ref):
        sm = sm_ref[...]
        b = lax.bitcast_convert_type(sm, jnp.uint32)
        key = b ^ jnp.where((b >> 31) != 0,
                            jnp.uint32(0xFFFFFFFF), jnp.uint32(0x80000000))
        kf = jnp.float32(k)

        def bs(t, T):
            cand = T | (jnp.uint32(1) << (jnp.uint32(31) - jnp.uint32(t)))
            cnt = jnp.sum((key >= cand).astype(jnp.float32))
            return jnp.where(cnt >= kf, cand, T)
        T = lax.fori_loop(0, 32, bs, jnp.uint32(0))
        gt = key > T
        r = kf - jnp.sum(gt.astype(jnp.float32))
        tie = (key == T).astype(jnp.float32)
        i0 = lax.broadcasted_iota(jnp.int32, (128, 128), 0)
        i1 = lax.broadcasted_iota(jnp.int32, (128, 128), 1)
        lt = (i0 <= i1).astype(jnp.float32)
        rowc = jnp.dot(tie, lt, preferred_element_type=jnp.float32)
        rs = rowc[:, 127:128]
        j0 = lax.broadcasted_iota(jnp.int32, (80, 80), 0)
        j1 = lax.broadcasted_iota(jnp.int32, (80, 80), 1)
        slt = (j1 < j0).astype(jnp.float32)
        offs = jnp.dot(slt, rs, preferred_element_type=jnp.float32)
        pc = rowc + offs
        keep = jnp.where(gt | ((key == T) & (pc <= r)), 1.0, 0.0)
        keep_ref[...] = keep
        scl_ref[...] = sm * keep

    return pl.pallas_call(
        body,
        out_shape=[
            jax.ShapeDtypeStruct((80, 128), jnp.float32),
            jax.ShapeDtypeStruct((80, 128), jnp.float32),
        ],
    )(sm80)


def _scale_pad(h, scl, keep):
    BN = 400

    def body(h_ref, s_ref, k_ref, o_ref):
        v = jnp.maximum(h_ref[...] * s_ref[...], 0.0)
        li = lax.broadcasted_iota(jnp.int32, (BN, 128), 1)
        fsel = jnp.where((li >= 64) & (li < 80), 1.0, 0.0)
        o_ref[...] = v + k_ref[...] * fsel

    return pl.pallas_call(
        body,
        grid=(N // BN,),
        in_specs=[
            pl.BlockSpec((BN, 128), lambda j: (j, 0)),
            pl.BlockSpec((BN, 1), lambda j: (j, 0)),
            pl.BlockSpec((BN, 1), lambda j: (j, 0)),
        ],
        out_specs=pl.BlockSpec((BN, 128), lambda j: (j, 0)),
        out_shape=jax.ShapeDtypeStruct((N, 128), jnp.float32),
    )(h, scl, keep)


def _scale_reduce(h, scl):
    BN = 400
    G = N // BN

    def body(h_ref, s_ref, g_ref):
        j = pl.program_id(0)
        v = jnp.maximum(h_ref[...] * s_ref[...], 0.0)

        @pl.when(j == 0)
        def _():
            g_ref[...] = jnp.zeros((1, 128), jnp.float32)
        g_ref[...] += jnp.sum(v, axis=0, keepdims=True)

    return pl.pallas_call(
        body,
        grid=(G,),
        in_specs=[
            pl.BlockSpec((BN, 128), lambda j: (j, 0)),
            pl.BlockSpec((BN, 1), lambda j: (j, 0)),
        ],
        out_specs=pl.BlockSpec((1, 128), lambda j: (0, 0)),
        out_shape=jax.ShapeDtypeStruct((1, 128), jnp.float32),
    )(h, scl)


def _head(g, w1, b1, w2, b2):
    def body(g_ref, w1r, b1r, w2r, b2r, o_ref):
        gg = g_ref[...] * jnp.float32(1.0 / 80.0)
        z1 = jnp.dot(gg, w1r[...], preferred_element_type=jnp.float32) + b1r[...]
        z = jnp.dot(z1, w2r[...], preferred_element_type=jnp.float32) + b2r[...]
        mx = jnp.max(z, axis=1, keepdims=True)
        o_ref[...] = z - (mx + jnp.log(jnp.sum(jnp.exp(z - mx), axis=1,
                                               keepdims=True)))

    return pl.pallas_call(
        body,
        out_shape=jax.ShapeDtypeStruct((1, 10), jnp.float32),
    )(g, w1, b1, w2, b2)


def _pad_tail(w, b, wn):
    dh = w.shape[0]
    wp = jnp.zeros((dh, 128), jnp.float32).at[:, :64].set(w)
    bp = jnp.zeros((128,), jnp.float32).at[:64].set(b)
    nrm = wn / (jnp.linalg.norm(wn) + 1e-16)
    np_ = jnp.zeros((128,), jnp.float32).at[:wn.shape[0]].set(nrm)
    return wp, bp[None, :], np_[:, None]


def _layer(plane, es, src, dst, cp, pw, k, alive_col, din, dh, conv, pad_tail):
    beta16 = jnp.full((16,), cp['beta'], jnp.float32)
    accs = conv(plane, es, src, dst, beta16).reshape(NC, NACC, 128)
    if pad_tail:
        w2, b2, wn = _pad_tail(cp['W2'], cp['b2'], pw)
    else:
        w2, b2 = cp['W2'], cp['b2'][None, :]
        wn = (pw / (jnp.linalg.norm(pw) + 1e-16))[:, None]
    h, sm = _mlp_score(plane, accs, cp['W1'], cp['b1'][None, :], w2, b2,
                       wn, alive_col, din, dh)
    sm80 = jnp.pad(sm[:, 0], (0, 240), constant_values=-3.0).reshape(80, 128)
    keep, scl = _pool_mask(sm80, k)
    keep_col = keep.reshape(-1)[:N, None]
    scl_col = scl.reshape(-1)[:N, None]
    return h, scl_col, keep_col


def kernel(x, edge_index, edge_attr, batch, params):
    src = edge_index[0]
    dst = edge_index[1]
    ones = jnp.ones((N, 1), jnp.float32)

    es1 = _edge_feats(edge_attr, params['c1']['We'])
    h1, scl1, keep1 = _layer(x, es1, src, dst, params['c1'], params['p1'],
                             2000, ones, 128, 256, _sc_conv_a, True)

    we2 = jnp.zeros((16, 128), jnp.float32).at[:, :64].set(params['c2']['We'])
    plane2 = _scale_pad(h1, scl1, keep1)
    es2 = _edge_feats(edge_attr, we2)
    h2, scl2, keep2 = _layer(plane2, es2, src, dst, params['c2'], params['p2'],
                             400, keep1, 64, 128, _sc_conv_b, True)

    we3 = jnp.zeros((16, 128), jnp.float32).at[:, :64].set(params['c3']['We'])
    plane3 = _scale_pad(h2, scl2, keep2)
    es3 = _edge_feats(edge_attr, we3)
    h3, scl3, keep3 = _layer(plane3, es3, src, dst, params['c3'], params['p3'],
                             80, keep2, 64, 128, _sc_conv_b, False)

    g = _scale_reduce(h3, scl3)
    return _head(g, params['d1W'], params['d1b'][None, :],
                 params['d2W'], params['d2b'][None, :])


# speedup vs baseline: 6.7931x; 1.0958x over previous
import jax
import jax.numpy as jnp
from jax import lax
from jax.experimental import pallas as pl
from jax.experimental.pallas import tpu as pltpu
from jax.experimental.pallas import tpu_sc as plsc

N = 10000
E = 160000
NACC = 10112
EPS = 1e-7
NC, NS = 2, 16
RZ = NACC // NS


def _sc_mesh():
    return plsc.VectorSubcoreMesh(
        core_axis_name="c", subcore_axis_name="s", num_cores=NC, num_subcores=NS
    )


BE_A = 80
EPT_A = E // NS
NBLK_A = EPT_A // BE_A


def _sc_conv_a_body(xs, es, srcv, dstv, beta, out,
                    acc, srcb, dstb, xb, eb, ob, zb, bb, sx0, sx1, se0, se1):
    c = lax.axis_index("c")
    s = lax.axis_index("s")
    zeros16 = jnp.zeros((16,), jnp.float32)
    sx = (sx0, sx1)
    se = (se0, se1)

    def zrow(r, _):
        for k in range(8):
            zb[r, pl.ds(16 * k, 16)] = zeros16
        return 0
    lax.fori_loop(0, 8, zrow, 0)

    def zcp(b, _):
        pltpu.sync_copy(zb, acc.at[pl.ds(s * RZ + b * 8, 8)])
        return 0
    lax.fori_loop(0, RZ // 8, zcp, 0)
    pltpu.sync_copy(beta, bb)
    plsc.subcore_barrier()
    bv = bb[...]

    def issue(b, bi):
        off = s * EPT_A + bi * BE_A
        pltpu.sync_copy(srcv.at[pl.ds(off, BE_A)], srcb.at[b])
        pltpu.async_copy(xs.at[srcb.at[b]], xb.at[b], sx[b])
        pltpu.async_copy(es.at[pl.ds(off, BE_A)], eb.at[b], se[b])
        pltpu.sync_copy(dstv.at[pl.ds(off, BE_A // 2)], dstb.at[b, 0])
        pltpu.sync_copy(dstv.at[pl.ds(off + BE_A // 2, BE_A // 2)],
                        dstb.at[b, 1])

    def compute(b):
        pltpu.make_async_copy(xs.at[srcb.at[b]], xb.at[b], sx[b]).wait()
        pltpu.make_async_copy(es.at[pl.ds(0, BE_A)], eb.at[b], se[b]).wait()
        for h in range(2):
            def row(i, _):
                for u in range(4):
                    jo = 4 * i + u
                    j = h * (BE_A // 2) + jo
                    for k in range(4):
                        col = 16 * k
                        xk = xb[b, j, pl.ds(c * 64 + col, 16)]
                        ek = eb[b, j, pl.ds(c * 64 + col, 16)]
                        m = jnp.maximum(xk + ek, 0.0) + EPS
                        w = jnp.exp(bv * m)
                        ob[jo, pl.ds(col, 16)] = w * m
                        ob[jo, pl.ds(64 + col, 16)] = w
                return 0
            lax.fori_loop(0, BE_A // 8, row, 0)
            pltpu.sync_copy(ob, acc.at[dstb.at[b, h]], add=True)

    issue(0, 0)

    def pair(t, _):
        issue(1, 2 * t + 1)
        compute(0)
        issue(0, 2 * t + 2)
        compute(1)
        return 0
    lax.fori_loop(0, (NBLK_A - 1) // 2, pair, 0)
    compute(0)
    plsc.subcore_barrier()
    pltpu.sync_copy(acc.at[pl.ds(s * RZ, RZ)],
                    out.at[pl.ds(c * NACC + s * RZ, RZ)])


def _make_sc_conv_a():
    return pl.kernel(
        _sc_conv_a_body,
        out_type=jax.ShapeDtypeStruct((NC * NACC, 128), jnp.float32),
        mesh=_sc_mesh(),
        scratch_types=[
            pltpu.VMEM_SHARED((NACC, 128), jnp.float32),
            pltpu.VMEM((2, BE_A), jnp.int32),
            pltpu.VMEM((2, 2, BE_A // 2), jnp.int32),
            pltpu.VMEM((2, BE_A, 128), jnp.float32),
            pltpu.VMEM((2, BE_A, 128), jnp.float32),
            pltpu.VMEM((BE_A // 2, 128), jnp.float32),
            pltpu.VMEM((8, 128), jnp.float32),
            pltpu.VMEM((16,), jnp.float32),
            pltpu.SemaphoreType.DMA,
            pltpu.SemaphoreType.DMA,
            pltpu.SemaphoreType.DMA,
            pltpu.SemaphoreType.DMA,
        ],
    )


BE_B = 40
E2 = E // NC
EPT_B = E2 // NS
NBLK_B = EPT_B // BE_B


def _sc_conv_b_body(xs, es, srcv, dstv, beta, out,
                    acc, srcb, dstb, xb, eb, ob, zb, bb, sx0, sx1, se0, se1):
    c = lax.axis_index("c")
    s = lax.axis_index("s")
    zeros16 = jnp.zeros((16,), jnp.float32)
    sx = (sx0, sx1)
    se = (se0, se1)

    def zrow(r, _):
        for k in range(8):
            zb[r, pl.ds(16 * k, 16)] = zeros16
        return 0
    lax.fori_loop(0, 8, zrow, 0)

    def zcp(b, _):
        pltpu.sync_copy(zb, acc.at[pl.ds(s * RZ + b * 8, 8)])
        return 0
    lax.fori_loop(0, RZ // 8, zcp, 0)
    pltpu.sync_copy(beta, bb)
    plsc.subcore_barrier()
    bv = bb[...]

    def issue(b, bi):
        off = c * E2 + s * EPT_B + bi * BE_B
        pltpu.sync_copy(srcv.at[pl.ds(off, BE_B)], srcb.at[b])
        pltpu.async_copy(xs.at[srcb.at[b]], xb.at[b], sx[b])
        pltpu.async_copy(es.at[pl.ds(off, BE_B)], eb.at[b], se[b])
        pltpu.sync_copy(dstv.at[pl.ds(off, BE_B)], dstb.at[b])

    def compute(b):
        pltpu.make_async_copy(xs.at[srcb.at[b]], xb.at[b], sx[b]).wait()
        pltpu.make_async_copy(es.at[pl.ds(0, BE_B)], eb.at[b], se[b]).wait()

        def row(i, _):
            for u in range(4):
                j = 4 * i + u
                f = xb[b, j, pl.ds(64, 16)]
                for k in range(4):
                    col = 16 * k
                    xk = xb[b, j, pl.ds(col, 16)]
                    ek = eb[b, j, pl.ds(col, 16)]
                    m = jnp.maximum(xk + ek, 0.0) + EPS
                    wf = jnp.exp(bv * m) * f
                    ob[j, pl.ds(col, 16)] = wf * m
                    ob[j, pl.ds(64 + col, 16)] = wf
            return 0
        lax.fori_loop(0, BE_B // 4, row, 0)
        pltpu.sync_copy(ob, acc.at[dstb.at[b]], add=True)

    issue(0, 0)

    def pair(t, _):
        issue(1, 2 * t + 1)
        compute(0)
        issue(0, 2 * t + 2)
        compute(1)
        return 0
    lax.fori_loop(0, (NBLK_B - 1) // 2, pair, 0)
    compute(0)
    plsc.subcore_barrier()
    pltpu.sync_copy(acc.at[pl.ds(s * RZ, RZ)],
                    out.at[pl.ds(c * NACC + s * RZ, RZ)])


def _make_sc_conv_b():
    return pl.kernel(
        _sc_conv_b_body,
        out_type=jax.ShapeDtypeStruct((NC * NACC, 128), jnp.float32),
        mesh=_sc_mesh(),
        scratch_types=[
            pltpu.VMEM_SHARED((NACC, 128), jnp.float32),
            pltpu.VMEM((2, BE_B), jnp.int32),
            pltpu.VMEM((2, BE_B), jnp.int32),
            pltpu.VMEM((2, BE_B, 128), jnp.float32),
            pltpu.VMEM((2, BE_B, 128), jnp.float32),
            pltpu.VMEM((BE_B, 128), jnp.float32),
            pltpu.VMEM((8, 128), jnp.float32),
            pltpu.VMEM((16,), jnp.float32),
            pltpu.SemaphoreType.DMA,
            pltpu.SemaphoreType.DMA,
            pltpu.SemaphoreType.DMA,
            pltpu.SemaphoreType.DMA,
        ],
    )


_sc_conv_a = _make_sc_conv_a()
_sc_conv_b = _make_sc_conv_b()


def _edge_feats(ea, w):
    R, din16 = ea.shape
    BR = 2000

    def body(ea_ref, w_ref, o_ref):
        o_ref[...] = jnp.dot(ea_ref[...], w_ref[...],
                             preferred_element_type=jnp.float32)

    return pl.pallas_call(
        body,
        grid=(R // BR,),
        in_specs=[
            pl.BlockSpec((BR, din16), lambda j: (j, 0)),
            pl.BlockSpec((din16, 128), lambda j: (0, 0)),
        ],
        out_specs=pl.BlockSpec((BR, 128), lambda j: (j, 0)),
        out_shape=jax.ShapeDtypeStruct((R, 128), jnp.float32),
    )(ea, w)


def _mlp_score(plane, accs, w1, b1, w2, b2, wn, alv, din, dh):
    BN = 400
    G = N // BN

    def body(x_ref, a0, a1, w1r, b1r, w2r, b2r, wnr, ar, h_ref, sm_ref):
        if din == 128:
            x = x_ref[...]
            ws = jnp.concatenate((a0[0][:, :64], a1[0][:, :64]), axis=1)
            ss = jnp.concatenate((a0[0][:, 64:], a1[0][:, 64:]), axis=1)
        else:
            x = x_ref[...][:, :64]
            ws = a0[0][:, :64] + a1[0][:, :64]
            ss = a0[0][:, 64:] + a1[0][:, 64:]
        h = x + ws / (ss + 1e-16)
        h1 = jnp.maximum(
            jnp.dot(h, w1r[...], preferred_element_type=jnp.float32) + b1r[...],
            0.0)
        h2 = jnp.dot(h1, w2r[...], preferred_element_type=jnp.float32) + b2r[...]
        h_ref[...] = h2
        sc = jnp.tanh(jnp.dot(h2, wnr[...], preferred_element_type=jnp.float32))
        sm_ref[...] = jnp.where(ar[...] > 0.0, sc, -2.0)

    return pl.pallas_call(
        body,
        grid=(G,),
        in_specs=[
            pl.BlockSpec((BN, 128), lambda j: (j, 0)),
            pl.BlockSpec((1, BN, 128), lambda j: (0, j, 0)),
            pl.BlockSpec((1, BN, 128), lambda j: (1, j, 0)),
            pl.BlockSpec((din, dh), lambda j: (0, 0)),
            pl.BlockSpec((1, dh), lambda j: (0, 0)),
            pl.BlockSpec((dh, 128), lambda j: (0, 0)),
            pl.BlockSpec((1, 128), lambda j: (0, 0)),
            pl.BlockSpec((128, 1), lambda j: (0, 0)),
            pl.BlockSpec((BN, 1), lambda j: (j, 0)),
        ],
        out_specs=[
            pl.BlockSpec((BN, 128), lambda j: (j, 0)),
            pl.BlockSpec((BN, 1), lambda j: (j, 0)),
        ],
        out_shape=[
            jax.ShapeDtypeStruct((N, 128), jnp.float32),
            jax.ShapeDtypeStruct((N, 1), jnp.float32),
        ],
    )(plane, accs, accs, w1, b1, w2, b2, wn, alv)


def _pool_mask(sm80, k):
    def body(sm_ref, keep_ref, scl_ref):
        sm = sm_ref[...]
        b = lax.bitcast_convert_type(sm, jnp.uint32)
        key = b ^ jnp.where((b >> 31) != 0,
                            jnp.uint32(0xFFFFFFFF), jnp.uint32(0x80000000))
        kf = jnp.float32(k)

        def bs(t, T):
            cand = T | (jnp.uint32(1) << (jnp.uint32(31) - jnp.uint32(t)))
            cnt = jnp.sum((key >= cand).astype(jnp.float32))
            return jnp.where(cnt >= kf, cand, T)
        T = lax.fori_loop(0, 32, bs, jnp.uint32(0))
        gt = key > T
        r = kf - jnp.sum(gt.astype(jnp.float32))
        tie = (key == T).astype(jnp.float32)
        i0 = lax.broadcasted_iota(jnp.int32, (128, 128), 0)
        i1 = lax.broadcasted_iota(jnp.int32, (128, 128), 1)
        lt = (i0 <= i1).astype(jnp.float32)
        rowc = jnp.dot(tie, lt, preferred_element_type=jnp.float32)
        rs = rowc[:, 127:128]
        j0 = lax.broadcasted_iota(jnp.int32, (80, 80), 0)
        j1 = lax.broadcasted_iota(jnp.int32, (80, 80), 1)
        slt = (j1 < j0).astype(jnp.float32)
        offs = jnp.dot(slt, rs, preferred_element_type=jnp.float32)
        pc = rowc + offs
        keep = jnp.where(gt | ((key == T) & (pc <= r)), 1.0, 0.0)
        keep_ref[...] = keep
        scl_ref[...] = sm * keep

    return pl.pallas_call(
        body,
        out_shape=[
            jax.ShapeDtypeStruct((80, 128), jnp.float32),
            jax.ShapeDtypeStruct((80, 128), jnp.float32),
        ],
    )(sm80)


def _scale_pad(h, scl, keep):
    BN = 400

    def body(h_ref, s_ref, k_ref, o_ref):
        v = jnp.maximum(h_ref[...] * s_ref[...], 0.0)
        li = lax.broadcasted_iota(jnp.int32, (BN, 128), 1)
        fsel = jnp.where((li >= 64) & (li < 80), 1.0, 0.0)
        o_ref[...] = v + k_ref[...] * fsel

    return pl.pallas_call(
        body,
        grid=(N // BN,),
        in_specs=[
            pl.BlockSpec((BN, 128), lambda j: (j, 0)),
            pl.BlockSpec((BN, 1), lambda j: (j, 0)),
            pl.BlockSpec((BN, 1), lambda j: (j, 0)),
        ],
        out_specs=pl.BlockSpec((BN, 128), lambda j: (j, 0)),
        out_shape=jax.ShapeDtypeStruct((N, 128), jnp.float32),
    )(h, scl, keep)


def _scale_reduce(h, scl):
    BN = 400
    G = N // BN

    def body(h_ref, s_ref, g_ref):
        j = pl.program_id(0)
        v = jnp.maximum(h_ref[...] * s_ref[...], 0.0)

        @pl.when(j == 0)
        def _():
            g_ref[...] = jnp.zeros((1, 128), jnp.float32)
        g_ref[...] += jnp.sum(v, axis=0, keepdims=True)

    return pl.pallas_call(
        body,
        grid=(G,),
        in_specs=[
            pl.BlockSpec((BN, 128), lambda j: (j, 0)),
            pl.BlockSpec((BN, 1), lambda j: (j, 0)),
        ],
        out_specs=pl.BlockSpec((1, 128), lambda j: (0, 0)),
        out_shape=jax.ShapeDtypeStruct((1, 128), jnp.float32),
    )(h, scl)


def _head(g, w1, b1, w2, b2):
    def body(g_ref, w1r, b1r, w2r, b2r, o_ref):
        gg = g_ref[...] * jnp.float32(1.0 / 80.0)
        z1 = jnp.dot(gg, w1r[...], preferred_element_type=jnp.float32) + b1r[...]
        z = jnp.dot(z1, w2r[...], preferred_element_type=jnp.float32) + b2r[...]
        mx = jnp.max(z, axis=1, keepdims=True)
        o_ref[...] = z - (mx + jnp.log(jnp.sum(jnp.exp(z - mx), axis=1,
                                               keepdims=True)))

    return pl.pallas_call(
        body,
        out_shape=jax.ShapeDtypeStruct((1, 10), jnp.float32),
    )(g, w1, b1, w2, b2)


def _pad_tail(w, b, wn):
    dh = w.shape[0]
    wp = jnp.zeros((dh, 128), jnp.float32).at[:, :64].set(w)
    bp = jnp.zeros((128,), jnp.float32).at[:64].set(b)
    nrm = wn / (jnp.linalg.norm(wn) + 1e-16)
    np_ = jnp.zeros((128,), jnp.float32).at[:wn.shape[0]].set(nrm)
    return wp, bp[None, :], np_[:, None]


def _layer(plane, es, src, dst, cp, pw, k, alive_col, din, dh, conv, pad_tail):
    beta16 = jnp.full((16,), cp['beta'], jnp.float32)
    accs = conv(plane, es, src, dst, beta16).reshape(NC, NACC, 128)
    if pad_tail:
        w2, b2, wn = _pad_tail(cp['W2'], cp['b2'], pw)
    else:
        w2, b2 = cp['W2'], cp['b2'][None, :]
        wn = (pw / (jnp.linalg.norm(pw) + 1e-16))[:, None]
    h, sm = _mlp_score(plane, accs, cp['W1'], cp['b1'][None, :], w2, b2,
                       wn, alive_col, din, dh)
    sm80 = jnp.pad(sm[:, 0], (0, 240), constant_values=-3.0).reshape(80, 128)
    keep, scl = _pool_mask(sm80, k)
    keep_col = keep.reshape(-1)[:N, None]
    scl_col = scl.reshape(-1)[:N, None]
    return h, scl_col, keep_col


def kernel(x, edge_index, edge_attr, batch, params):
    src = edge_index[0]
    dst = edge_index[1]
    ones = jnp.ones((N, 1), jnp.float32)

    es1 = _edge_feats(edge_attr, params['c1']['We'])
    h1, scl1, keep1 = _layer(x, es1, src, dst, params['c1'], params['p1'],
                             2000, ones, 128, 256, _sc_conv_a, True)

    we2 = jnp.zeros((16, 128), jnp.float32).at[:, :64].set(params['c2']['We'])
    plane2 = _scale_pad(h1, scl1, keep1)
    es2 = _edge_feats(edge_attr, we2)
    h2, scl2, keep2 = _layer(plane2, es2, src, dst, params['c2'], params['p2'],
                             400, keep1, 64, 128, _sc_conv_b, True)

    we3 = jnp.zeros((16, 128), jnp.float32).at[:, :64].set(params['c3']['We'])
    plane3 = _scale_pad(h2, scl2, keep2)
    es3 = _edge_feats(edge_attr, we3)
    h3, scl3, keep3 = _layer(plane3, es3, src, dst, params['c3'], params['p3'],
                             80, keep2, 64, 128, _sc_conv_b, False)

    g = _scale_reduce(h3, scl3)
    return _head(g, params['d1W'], params['d1b'][None, :],
                 params['d2W'], params['d2b'][None, :])


# speedup vs baseline: 7.1467x; 1.0521x over previous
import jax
import jax.numpy as jnp
from jax import lax
from jax.experimental import pallas as pl
from jax.experimental.pallas import tpu as pltpu
from jax.experimental.pallas import tpu_sc as plsc

N = 10000
E = 160000
NACC = 10112
EPS = 1e-7
NC, NS = 2, 16
RZ = NACC // NS


def _sc_mesh():
    return plsc.VectorSubcoreMesh(
        core_axis_name="c", subcore_axis_name="s", num_cores=NC, num_subcores=NS
    )


BE_A = 80
EPT_A = E // NS
NBLK_A = EPT_A // BE_A
HB_A = BE_A // 2


def _zero_acc(acc, zb, s, zsem):
    zeros16 = jnp.zeros((16,), jnp.float32)

    def zrow(r, _):
        for k in range(8):
            zb[r, pl.ds(16 * k, 16)] = zeros16
        return 0
    lax.fori_loop(0, 8, zrow, 0)

    def zcp(b, _):
        pltpu.async_copy(zb, acc.at[pl.ds(s * RZ + b * 8, 8)], zsem)
        return 0

    def zwait(b, _):
        pltpu.make_async_copy(zb, acc.at[pl.ds(s * RZ, 8)], zsem).wait()
        return 0
    lax.fori_loop(0, RZ // 8, zcp, 0)
    lax.fori_loop(0, RZ // 8, zwait, 0)


def _sc_conv_a_body(xs, es, srcv, dstv, beta, out,
                    acc, srcb, dstb, scb, xb, eb, ob, zb, bb,
                    sx0, sx1, se0, se1, so0, so1):
    c = lax.axis_index("c")
    s = lax.axis_index("s")
    sx = (sx0, sx1)
    se = (se0, se1)
    so = (so0, so1)

    _zero_acc(acc, zb, s, so0)
    pltpu.sync_copy(beta, bb)
    plsc.subcore_barrier()
    bv = bb[...]

    def issue(b, bi):
        off = s * EPT_A + bi * BE_A
        pltpu.sync_copy(srcv.at[pl.ds(off, BE_A)], srcb.at[b])
        pltpu.async_copy(xs.at[srcb.at[b]], xb.at[b], sx[b])
        eoff = pl.multiple_of(c * (E // 2) + off // 2, 8)
        pltpu.async_copy(es.at[pl.ds(eoff, HB_A)], eb.at[b], se[b])
        pltpu.sync_copy(dstv.at[pl.ds(off, HB_A)], dstb.at[b, 0, pl.ds(0, HB_A)])
        pltpu.sync_copy(dstv.at[pl.ds(off + HB_A, HB_A)],
                        dstb.at[b, 1, pl.ds(0, HB_A)])

    def compute(b, do_drain):
        pltpu.make_async_copy(xs.at[srcb.at[b]], xb.at[b], sx[b]).wait()
        pltpu.make_async_copy(es.at[pl.ds(0, HB_A)], eb.at[b], se[b]).wait()
        for h in range(2):
            @pl.when(do_drain)
            def _():
                pltpu.make_async_copy(ob.at[h],
                                      acc.at[scb.at[h, pl.ds(0, HB_A)]],
                                      so[h]).wait()
            for kk in range(3):
                scb[h, pl.ds(16 * kk, 16)] = dstb[b, h, pl.ds(16 * kk, 16)]

            def row(i, _):
                for u in range(2):
                    jo = 2 * i + u
                    jh = h * (HB_A // 2) + jo
                    for p in range(2):
                        j = 2 * jo + p
                        for k in range(4):
                            col = 16 * k
                            xk = xb[b, 2 * jh + p, pl.ds(c * 64 + col, 16)]
                            ek = eb[b, jh, pl.ds(64 * p + col, 16)]
                            m = jnp.maximum(xk + ek, 0.0) + EPS
                            w = jnp.exp(bv * m)
                            ob[h, j, pl.ds(col, 16)] = w * m
                            ob[h, j, pl.ds(64 + col, 16)] = w
                return 0
            lax.fori_loop(0, HB_A // 4, row, 0)
            pltpu.async_copy(ob.at[h], acc.at[scb.at[h, pl.ds(0, HB_A)]],
                             so[h], add=True)

    issue(0, 0)

    def pair(t, _):
        issue(1, 2 * t + 1)
        compute(0, t > 0)
        issue(0, 2 * t + 2)
        compute(1, t >= 0)
        return 0
    lax.fori_loop(0, (NBLK_A - 1) // 2, pair, 0)
    compute(0, jnp.bool_(True))
    pltpu.make_async_copy(ob.at[0], acc.at[scb.at[0, pl.ds(0, HB_A)]],
                          so[0]).wait()
    pltpu.make_async_copy(ob.at[1], acc.at[scb.at[1, pl.ds(0, HB_A)]],
                          so[1]).wait()
    plsc.subcore_barrier()
    pltpu.sync_copy(acc.at[pl.ds(s * RZ, RZ)],
                    out.at[pl.ds(c * NACC + s * RZ, RZ)])


def _make_sc_conv_a():
    return pl.kernel(
        _sc_conv_a_body,
        out_type=jax.ShapeDtypeStruct((NC * NACC, 128), jnp.float32),
        mesh=_sc_mesh(),
        scratch_types=[
            pltpu.VMEM_SHARED((NACC, 128), jnp.float32),
            pltpu.VMEM((2, BE_A), jnp.int32),
            pltpu.VMEM((2, 2, 48), jnp.int32),
            pltpu.VMEM((2, 48), jnp.int32),
            pltpu.VMEM((2, BE_A, 128), jnp.float32),
            pltpu.VMEM((2, HB_A, 128), jnp.float32),
            pltpu.VMEM((2, HB_A, 128), jnp.float32),
            pltpu.VMEM((8, 128), jnp.float32),
            pltpu.VMEM((16,), jnp.float32),
            pltpu.SemaphoreType.DMA,
            pltpu.SemaphoreType.DMA,
            pltpu.SemaphoreType.DMA,
            pltpu.SemaphoreType.DMA,
            pltpu.SemaphoreType.DMA,
            pltpu.SemaphoreType.DMA,
        ],
    )


BE_B = 40
E2 = E // NC
EPT_B = E2 // NS
NBLK_B = EPT_B // BE_B


def _sc_conv_b_body(xs, es, srcv, dstv, beta, out,
                    acc, srcb, dstb, xb, eb, ob, zb, bb, sx0, sx1, se0, se1):
    c = lax.axis_index("c")
    s = lax.axis_index("s")
    zeros16 = jnp.zeros((16,), jnp.float32)
    sx = (sx0, sx1)
    se = (se0, se1)

    def zrow(r, _):
        for k in range(8):
            zb[r, pl.ds(16 * k, 16)] = zeros16
        return 0
    lax.fori_loop(0, 8, zrow, 0)

    def zcp(b, _):
        pltpu.sync_copy(zb, acc.at[pl.ds(s * RZ + b * 8, 8)])
        return 0
    lax.fori_loop(0, RZ // 8, zcp, 0)
    pltpu.sync_copy(beta, bb)
    plsc.subcore_barrier()
    bv = bb[...]

    def issue(b, bi):
        off = c * E2 + s * EPT_B + bi * BE_B
        pltpu.sync_copy(srcv.at[pl.ds(off, BE_B)], srcb.at[b])
        pltpu.async_copy(xs.at[srcb.at[b]], xb.at[b], sx[b])
        pltpu.async_copy(es.at[pl.ds(off, BE_B)], eb.at[b], se[b])
        pltpu.sync_copy(dstv.at[pl.ds(off, BE_B)], dstb.at[b])

    def compute(b):
        pltpu.make_async_copy(xs.at[srcb.at[b]], xb.at[b], sx[b]).wait()
        pltpu.make_async_copy(es.at[pl.ds(0, BE_B)], eb.at[b], se[b]).wait()

        def row(i, _):
            for u in range(4):
                j = 4 * i + u
                f = xb[b, j, pl.ds(64, 16)]
                for k in range(4):
                    col = 16 * k
                    xk = xb[b, j, pl.ds(col, 16)]
                    ek = eb[b, j, pl.ds(col, 16)]
                    m = jnp.maximum(xk + ek, 0.0) + EPS
                    wf = jnp.exp(bv * m) * f
                    ob[j, pl.ds(col, 16)] = wf * m
                    ob[j, pl.ds(64 + col, 16)] = wf
            return 0
        lax.fori_loop(0, BE_B // 4, row, 0)
        pltpu.sync_copy(ob, acc.at[dstb.at[b]], add=True)

    issue(0, 0)

    def pair(t, _):
        issue(1, 2 * t + 1)
        compute(0)
        issue(0, 2 * t + 2)
        compute(1)
        return 0
    lax.fori_loop(0, (NBLK_B - 1) // 2, pair, 0)
    compute(0)
    plsc.subcore_barrier()
    pltpu.sync_copy(acc.at[pl.ds(s * RZ, RZ)],
                    out.at[pl.ds(c * NACC + s * RZ, RZ)])


def _make_sc_conv_b():
    return pl.kernel(
        _sc_conv_b_body,
        out_type=jax.ShapeDtypeStruct((NC * NACC, 128), jnp.float32),
        mesh=_sc_mesh(),
        scratch_types=[
            pltpu.VMEM_SHARED((NACC, 128), jnp.float32),
            pltpu.VMEM((2, BE_B), jnp.int32),
            pltpu.VMEM((2, BE_B), jnp.int32),
            pltpu.VMEM((2, BE_B, 128), jnp.float32),
            pltpu.VMEM((2, BE_B, 128), jnp.float32),
            pltpu.VMEM((BE_B, 128), jnp.float32),
            pltpu.VMEM((8, 128), jnp.float32),
            pltpu.VMEM((16,), jnp.float32),
            pltpu.SemaphoreType.DMA,
            pltpu.SemaphoreType.DMA,
            pltpu.SemaphoreType.DMA,
            pltpu.SemaphoreType.DMA,
        ],
    )


_sc_conv_a = _make_sc_conv_a()
_sc_conv_b = _make_sc_conv_b()


def _edge_feats(ea, w):
    R, din16 = ea.shape
    BR = 2000

    def body(ea_ref, w_ref, o_ref):
        o_ref[...] = jnp.dot(ea_ref[...], w_ref[...],
                             preferred_element_type=jnp.float32)

    return pl.pallas_call(
        body,
        grid=(R // BR,),
        in_specs=[
            pl.BlockSpec((BR, din16), lambda j: (j, 0)),
            pl.BlockSpec((din16, 128), lambda j: (0, 0)),
        ],
        out_specs=pl.BlockSpec((BR, 128), lambda j: (j, 0)),
        out_shape=jax.ShapeDtypeStruct((R, 128), jnp.float32),
    )(ea, w)


def _mlp_score(plane, accs, w1, b1, w2, b2, wn, alv, din, dh):
    BN = 400
    G = N // BN

    def body(x_ref, a0, a1, w1r, b1r, w2r, b2r, wnr, ar, h_ref, sm_ref):
        if din == 128:
            x = x_ref[...]
            ws = jnp.concatenate((a0[0][:, :64], a1[0][:, :64]), axis=1)
            ss = jnp.concatenate((a0[0][:, 64:], a1[0][:, 64:]), axis=1)
        else:
            x = x_ref[...][:, :64]
            ws = a0[0][:, :64] + a1[0][:, :64]
            ss = a0[0][:, 64:] + a1[0][:, 64:]
        h = x + ws / (ss + 1e-16)
        h1 = jnp.maximum(
            jnp.dot(h, w1r[...], preferred_element_type=jnp.float32) + b1r[...],
            0.0)
        h2 = jnp.dot(h1, w2r[...], preferred_element_type=jnp.float32) + b2r[...]
        h_ref[...] = h2
        sc = jnp.tanh(jnp.dot(h2, wnr[...], preferred_element_type=jnp.float32))
        sm_ref[...] = jnp.where(ar[...] > 0.0, sc, -2.0)

    return pl.pallas_call(
        body,
        grid=(G,),
        in_specs=[
            pl.BlockSpec((BN, 128), lambda j: (j, 0)),
            pl.BlockSpec((1, BN, 128), lambda j: (0, j, 0)),
            pl.BlockSpec((1, BN, 128), lambda j: (1, j, 0)),
            pl.BlockSpec((din, dh), lambda j: (0, 0)),
            pl.BlockSpec((1, dh), lambda j: (0, 0)),
            pl.BlockSpec((dh, 128), lambda j: (0, 0)),
            pl.BlockSpec((1, 128), lambda j: (0, 0)),
            pl.BlockSpec((128, 1), lambda j: (0, 0)),
            pl.BlockSpec((BN, 1), lambda j: (j, 0)),
        ],
        out_specs=[
            pl.BlockSpec((BN, 128), lambda j: (j, 0)),
            pl.BlockSpec((BN, 1), lambda j: (j, 0)),
        ],
        out_shape=[
            jax.ShapeDtypeStruct((N, 128), jnp.float32),
            jax.ShapeDtypeStruct((N, 1), jnp.float32),
        ],
    )(plane, accs, accs, w1, b1, w2, b2, wn, alv)


def _pool_mask(sm80, k):
    def body(sm_ref, keep_ref, scl_ref):
        sm = sm_ref[...]
        b = lax.bitcast_convert_type(sm, jnp.uint32)
        key = b ^ jnp.where((b >> 31) != 0,
                            jnp.uint32(0xFFFFFFFF), jnp.uint32(0x80000000))
        kf = jnp.float32(k)

        def bs(t, T):
            cand = T | (jnp.uint32(1) << (jnp.uint32(31) - jnp.uint32(t)))
            cnt = jnp.sum((key >= cand).astype(jnp.float32))
            return jnp.where(cnt >= kf, cand, T)
        T = lax.fori_loop(0, 32, bs, jnp.uint32(0))
        gt = key > T
        r = kf - jnp.sum(gt.astype(jnp.float32))
        tie = (key == T).astype(jnp.float32)
        i0 = lax.broadcasted_iota(jnp.int32, (128, 128), 0)
        i1 = lax.broadcasted_iota(jnp.int32, (128, 128), 1)
        lt = (i0 <= i1).astype(jnp.float32)
        rowc = jnp.dot(tie, lt, preferred_element_type=jnp.float32)
        rs = rowc[:, 127:128]
        j0 = lax.broadcasted_iota(jnp.int32, (80, 80), 0)
        j1 = lax.broadcasted_iota(jnp.int32, (80, 80), 1)
        slt = (j1 < j0).astype(jnp.float32)
        offs = jnp.dot(slt, rs, preferred_element_type=jnp.float32)
        pc = rowc + offs
        keep = jnp.where(gt | ((key == T) & (pc <= r)), 1.0, 0.0)
        keep_ref[...] = keep
        scl_ref[...] = sm * keep

    return pl.pallas_call(
        body,
        out_shape=[
            jax.ShapeDtypeStruct((80, 128), jnp.float32),
            jax.ShapeDtypeStruct((80, 128), jnp.float32),
        ],
    )(sm80)


def _scale_pad(h, scl, keep):
    BN = 400

    def body(h_ref, s_ref, k_ref, o_ref):
        v = jnp.maximum(h_ref[...] * s_ref[...], 0.0)
        li = lax.broadcasted_iota(jnp.int32, (BN, 128), 1)
        fsel = jnp.where((li >= 64) & (li < 80), 1.0, 0.0)
        o_ref[...] = v + k_ref[...] * fsel

    return pl.pallas_call(
        body,
        grid=(N // BN,),
        in_specs=[
            pl.BlockSpec((BN, 128), lambda j: (j, 0)),
            pl.BlockSpec((BN, 1), lambda j: (j, 0)),
            pl.BlockSpec((BN, 1), lambda j: (j, 0)),
        ],
        out_specs=pl.BlockSpec((BN, 128), lambda j: (j, 0)),
        out_shape=jax.ShapeDtypeStruct((N, 128), jnp.float32),
    )(h, scl, keep)


def _scale_reduce(h, scl):
    BN = 400
    G = N // BN

    def body(h_ref, s_ref, g_ref):
        j = pl.program_id(0)
        v = jnp.maximum(h_ref[...] * s_ref[...], 0.0)

        @pl.when(j == 0)
        def _():
            g_ref[...] = jnp.zeros((1, 128), jnp.float32)
        g_ref[...] += jnp.sum(v, axis=0, keepdims=True)

    return pl.pallas_call(
        body,
        grid=(G,),
        in_specs=[
            pl.BlockSpec((BN, 128), lambda j: (j, 0)),
            pl.BlockSpec((BN, 1), lambda j: (j, 0)),
        ],
        out_specs=pl.BlockSpec((1, 128), lambda j: (0, 0)),
        out_shape=jax.ShapeDtypeStruct((1, 128), jnp.float32),
    )(h, scl)


def _head(g, w1, b1, w2, b2):
    def body(g_ref, w1r, b1r, w2r, b2r, o_ref):
        gg = g_ref[...] * jnp.float32(1.0 / 80.0)
        z1 = jnp.dot(gg, w1r[...], preferred_element_type=jnp.float32) + b1r[...]
        z = jnp.dot(z1, w2r[...], preferred_element_type=jnp.float32) + b2r[...]
        mx = jnp.max(z, axis=1, keepdims=True)
        o_ref[...] = z - (mx + jnp.log(jnp.sum(jnp.exp(z - mx), axis=1,
                                               keepdims=True)))

    return pl.pallas_call(
        body,
        out_shape=jax.ShapeDtypeStruct((1, 10), jnp.float32),
    )(g, w1, b1, w2, b2)


def _pad_tail(w, b, wn):
    dh = w.shape[0]
    wp = jnp.zeros((dh, 128), jnp.float32).at[:, :64].set(w)
    bp = jnp.zeros((128,), jnp.float32).at[:64].set(b)
    nrm = wn / (jnp.linalg.norm(wn) + 1e-16)
    np_ = jnp.zeros((128,), jnp.float32).at[:wn.shape[0]].set(nrm)
    return wp, bp[None, :], np_[:, None]


def _layer(plane, es, src, dst, cp, pw, k, alive_col, din, dh, conv, pad_tail):
    beta16 = jnp.full((16,), cp['beta'], jnp.float32)
    accs = conv(plane, es, src, dst, beta16).reshape(NC, NACC, 128)
    if pad_tail:
        w2, b2, wn = _pad_tail(cp['W2'], cp['b2'], pw)
    else:
        w2, b2 = cp['W2'], cp['b2'][None, :]
        wn = (pw / (jnp.linalg.norm(pw) + 1e-16))[:, None]
    h, sm = _mlp_score(plane, accs, cp['W1'], cp['b1'][None, :], w2, b2,
                       wn, alive_col, din, dh)
    sm80 = jnp.pad(sm[:, 0], (0, 240), constant_values=-3.0).reshape(80, 128)
    keep, scl = _pool_mask(sm80, k)
    keep_col = keep.reshape(-1)[:N, None]
    scl_col = scl.reshape(-1)[:N, None]
    return h, scl_col, keep_col


def kernel(x, edge_index, edge_attr, batch, params):
    src = edge_index[0]
    dst = edge_index[1]
    ones = jnp.ones((N, 1), jnp.float32)

    es1 = _edge_feats(edge_attr, params['c1']['We'])
    h1, scl1, keep1 = _layer(x, es1, src, dst, params['c1'], params['p1'],
                             2000, ones, 128, 256, _sc_conv_a, True)

    we2 = jnp.zeros((16, 128), jnp.float32).at[:, :64].set(params['c2']['We'])
    plane2 = _scale_pad(h1, scl1, keep1)
    es2 = _edge_feats(edge_attr, we2)
    h2, scl2, keep2 = _layer(plane2, es2, src, dst, params['c2'], params['p2'],
                             400, keep1, 64, 128, _sc_conv_b, True)

    we3 = jnp.zeros((16, 128), jnp.float32).at[:, :64].set(params['c3']['We'])
    plane3 = _scale_pad(h2, scl2, keep2)
    es3 = _edge_feats(edge_attr, we3)
    h3, scl3, keep3 = _layer(plane3, es3, src, dst, params['c3'], params['p3'],
                             80, keep2, 64, 128, _sc_conv_b, False)

    g = _scale_reduce(h3, scl3)
    return _head(g, params['d1W'], params['d1b'][None, :],
                 params['d2W'], params['d2b'][None, :])
